# Initial kernel scaffold; baseline (speedup 1.0000x reference)
import jax
import jax.numpy as jnp
from jax.experimental import pallas as pl


def kernel(weights, nat_table, mov_table, cent_table, nat_idx, mov_idx, cent_idx):
    raise NotImplementedError("write your pallas kernel here")


# speedup vs baseline: 14.9883x; 14.9883x over previous
import functools

import jax
import jax.numpy as jnp
from jax import lax
from jax.experimental import pallas as pl
from jax.experimental.pallas import tpu as pltpu
from jax.experimental.pallas import tpu_sc as plsc

N_NAT = 30
N_MOV = 30
N_CENT = 9
B = 16384
M = 5
D_NAT = 64
D_MOV = 64
D_CENT = 32

NC = 2
NS = 16
L = 16
NW = NC * NS
BPW = B // NW
NV = BPW // L

ROW = 80
OFF_MOV = 32
OFF_CENT = 64
HIST_WORDS = L * ROW


def _sc_body(w_hbm, nat_hbm, mov_hbm, cent_hbm, ntab_hbm, mtab_hbm, ctab_hbm,
             part_hbm, wsum_hbm,
             w_v, nat_v, mov_v, cent_v, ntab_v, mtab_v, ctab_v,
             hist_v, svec_v, out_v, wsum_v):
    wid = lax.axis_index("s") * NC + lax.axis_index("c")
    base = wid * BPW

    pltpu.sync_copy(w_hbm.at[pl.ds(base, BPW)], w_v)
    pltpu.sync_copy(nat_hbm.at[pl.ds(base, BPW)], nat_v)
    for j in range(M):
        pltpu.sync_copy(mov_hbm.at[pl.ds(j * B + base, BPW)],
                        mov_v.at[pl.ds(j * BPW, BPW)])
    pltpu.sync_copy(cent_hbm.at[pl.ds(base, BPW)], cent_v)
    pltpu.sync_copy(ntab_hbm, ntab_v)
    pltpu.sync_copy(mtab_hbm, mtab_v)
    pltpu.sync_copy(ctab_hbm, ctab_v)

    zero = jnp.zeros((L,), jnp.float32)
    for i in range(HIST_WORDS // L):
        hist_v[pl.ds(i * L, L)] = zero

    rowbase = lax.iota(jnp.int32, L) * ROW
    wsum = zero
    for i in range(NV):
        o = i * L
        w = w_v[pl.ds(o, L)]
        wsum = wsum + w
        wm = w * jnp.float32(1.0 / M)
        ni = nat_v[pl.ds(o, L)]
        plsc.addupdate_scatter(hist_v, [rowbase + ni], w)
        for j in range(M):
            mj = mov_v[pl.ds(j * BPW + o, L)]
            plsc.addupdate_scatter(hist_v, [rowbase + (OFF_MOV + mj)], wm)
        ci = cent_v[pl.ds(o, L)]
        plsc.addupdate_scatter(hist_v, [rowbase + (OFF_CENT + ci)], w)

    for b in range(ROW // L):
        acc = hist_v[pl.ds(b * L, L)]
        for lane in range(1, L):
            acc = acc + hist_v[pl.ds(lane * ROW + b * L, L)]
        svec_v[pl.ds(b * L, L)] = acc

    accs = [jnp.zeros((L,), jnp.float32) for _ in range(10)]
    for k in range(N_NAT):
        sk = plsc.load_gather(svec_v, [jnp.full((L,), k, jnp.int32)])
        for cb in range(D_NAT // L):
            accs[cb] = accs[cb] + sk * ntab_v[pl.ds(k * D_NAT + cb * L, L)]
    for k in range(N_MOV):
        sk = plsc.load_gather(svec_v, [jnp.full((L,), OFF_MOV + k, jnp.int32)])
        for cb in range(D_MOV // L):
            accs[4 + cb] = accs[4 + cb] + sk * mtab_v[pl.ds(k * D_MOV + cb * L, L)]
    for k in range(N_CENT):
        sk = plsc.load_gather(svec_v, [jnp.full((L,), OFF_CENT + k, jnp.int32)])
        for cb in range(D_CENT // L):
            accs[8 + cb] = accs[8 + cb] + sk * ctab_v[pl.ds(k * D_CENT + cb * L, L)]

    for cb in range(10):
        out_v[pl.ds(cb * L, L)] = accs[cb]
    wsum_v[pl.ds(0, L)] = wsum
    pltpu.sync_copy(out_v, part_hbm.at[wid])
    pltpu.sync_copy(wsum_v, wsum_hbm.at[wid])


@jax.jit
def _run(weights, nat_idx, mov_flat, cent_idx, ntab, mtab, ctab):
    mesh = plsc.VectorSubcoreMesh(core_axis_name="c", subcore_axis_name="s",
                                  num_cores=NC, num_subcores=NS)
    part, wsum = pl.kernel(
        _sc_body,
        out_type=(jax.ShapeDtypeStruct((NW, 160), jnp.float32),
                  jax.ShapeDtypeStruct((NW, L), jnp.float32)),
        mesh=mesh,
        compiler_params=pltpu.CompilerParams(needs_layout_passes=False),
        scratch_types=[
            pltpu.VMEM((BPW,), jnp.float32),
            pltpu.VMEM((BPW,), jnp.int32),
            pltpu.VMEM((M * BPW,), jnp.int32),
            pltpu.VMEM((BPW,), jnp.int32),
            pltpu.VMEM((N_NAT * D_NAT,), jnp.float32),
            pltpu.VMEM((N_MOV * D_MOV,), jnp.float32),
            pltpu.VMEM((N_CENT * D_CENT,), jnp.float32),
            pltpu.VMEM((HIST_WORDS,), jnp.float32),
            pltpu.VMEM((ROW,), jnp.float32),
            pltpu.VMEM((160,), jnp.float32),
            pltpu.VMEM((L,), jnp.float32),
        ],
    )(weights, nat_idx, mov_flat, cent_idx, ntab, mtab, ctab)
    return part.sum(axis=0) / wsum.sum()


def kernel(weights, nat_table, mov_table, cent_table, nat_idx, mov_idx, cent_idx):
    nat_i = nat_idx.astype(jnp.int32)
    mov_flat = mov_idx.astype(jnp.int32).T.reshape(-1)
    cent_i = cent_idx.astype(jnp.int32)
    return _run(weights, nat_i, mov_flat, cent_i,
                nat_table.reshape(-1), mov_table.reshape(-1),
                cent_table.reshape(-1))
